# Initial kernel scaffold; baseline (speedup 1.0000x reference)
import jax
import jax.numpy as jnp
from jax.experimental import pallas as pl


def kernel(scores, k):
    raise NotImplementedError("write your pallas kernel here")


# speedup vs baseline: 130.9421x; 130.9421x over previous
import jax
import jax.numpy as jnp
from jax.experimental import pallas as pl
from jax.experimental.pallas import tpu as pltpu

_ROWS_PER_BLOCK = 8


def _body(k_ref, x_ref, o_ref):
    x = x_ref[...]
    bits = jax.lax.bitcast_convert_type(x, jnp.int32) & jnp.int32(0x7FFFFFFF)
    kk = k_ref[0]
    t = jnp.zeros((x.shape[0], 1), jnp.int32)
    for b in range(30, -1, -1):
        trial = t | jnp.int32(1 << b)
        cnt = jnp.sum((bits >= trial).astype(jnp.int32), axis=1, keepdims=True)
        t = jnp.where(cnt >= kk, trial, t)
    o_ref[...] = jnp.where(bits >= t, x, jnp.zeros_like(x))


def kernel(scores, k):
    rows, cols = scores.shape
    k_arr = jnp.asarray(k, jnp.int32).reshape(1)
    grid = rows // _ROWS_PER_BLOCK
    return pl.pallas_call(
        _body,
        grid=(grid,),
        in_specs=[
            pl.BlockSpec(memory_space=pltpu.SMEM),
            pl.BlockSpec((_ROWS_PER_BLOCK, cols), lambda i: (i, 0)),
        ],
        out_specs=pl.BlockSpec((_ROWS_PER_BLOCK, cols), lambda i: (i, 0)),
        out_shape=jax.ShapeDtypeStruct((rows, cols), scores.dtype),
    )(k_arr, scores)
